# baseline (speedup 1.0000x reference)
import functools

import jax
import jax.numpy as jnp
from jax import lax
from jax.experimental import pallas as pl
from jax.experimental.pallas import tpu as pltpu
from jax.experimental.pallas import tpu_sc as plsc

NSAMPLES = 1000000
NX = 32
BATCH = 16384


def kernel(ids, start_state):
    info = plsc.get_sparse_core_info()
    nc, ns = info.num_cores, info.num_subcores
    nw = nc * ns
    b_per_w = BATCH // nw

    mesh = plsc.VectorSubcoreMesh(core_axis_name="c", subcore_axis_name="s")

    @functools.partial(
        pl.kernel,
        mesh=mesh,
        out_type=jax.ShapeDtypeStruct((BATCH, NX), jnp.float32),
        compiler_params=pltpu.CompilerParams(use_tc_tiling_on_sc=False),
        scratch_types=[
            pltpu.VMEM((b_per_w,), jnp.int32),
            pltpu.VMEM((b_per_w, NX), jnp.float32),
            pltpu.SemaphoreType.DMA,
        ],
    )
    def _gather(table_hbm, idx_hbm, out_hbm, idx_v, rows_v, sem):
        wid = lax.axis_index("s") * nc + lax.axis_index("c")
        base = wid * b_per_w
        pltpu.sync_copy(idx_hbm.at[pl.ds(base, b_per_w)], idx_v)
        pltpu.async_copy(table_hbm.at[idx_v], rows_v, sem).wait()
        pltpu.sync_copy(rows_v, out_hbm.at[pl.ds(base, b_per_w)])

    return _gather(start_state, ids.astype(jnp.int32))


# speedup vs baseline: 3.4207x; 3.4207x over previous
import functools

import jax
import jax.numpy as jnp
from jax import lax
from jax.experimental import pallas as pl
from jax.experimental.pallas import tpu as pltpu
from jax.experimental.pallas import tpu_sc as plsc

NSAMPLES = 1000000
NX = 32
BATCH = 16384

SC_TILES = (NSAMPLES + 127) // 128
KT = 601
GROWS = SC_TILES * NX // 128
R_ROWS = 4 * SC_TILES * 8
NG = R_ROWS * 8
FR_G = SC_TILES * 64
CHUNK = 128


def _detile(tt):
    def body(in_ref, out_ref):
        x = in_ref[...]
        out_ref[...] = (
            x.reshape(8, KT, 128).swapaxes(0, 1).reshape(KT * 8, 128)
        )

    return pl.pallas_call(
        body,
        grid=(4, SC_TILES // KT),
        in_specs=[
            pl.BlockSpec((8, KT * 128), lambda fr, g: (fr, g)),
        ],
        out_specs=pl.BlockSpec((KT * 8, 128), lambda fr, g: (fr * (SC_TILES // KT) + g, 0)),
        out_shape=jax.ShapeDtypeStruct((R_ROWS, 128), jnp.float32),
    )(tt)


def kernel(ids, start_state):
    info = plsc.get_sparse_core_info()
    nc, ns = info.num_cores, info.num_subcores
    nw = nc * ns
    bw = BATCH // nw
    nch = bw // CHUNK

    mesh = plsc.VectorSubcoreMesh(core_axis_name="c", subcore_axis_name="s")

    @functools.partial(
        pl.kernel,
        mesh=mesh,
        out_type=jax.ShapeDtypeStruct((NX, BATCH), jnp.float32),
        compiler_params=pltpu.CompilerParams(
            use_tc_tiling_on_sc=False, needs_layout_passes=False
        ),
        scratch_types=[
            pltpu.VMEM((bw,), jnp.int32),
            pltpu.VMEM((NX * CHUNK,), jnp.int32),
            pltpu.VMEM((NX * CHUNK, 16), jnp.float32),
            pltpu.VMEM((NX, bw), jnp.float32),
            pltpu.SemaphoreType.DMA,
        ],
    )
    def _gather(rg_hbm, idx_hbm, out_hbm, ids_v, gi_v, rows_v, vals_v, sem):
        wid = lax.axis_index("s") * nc + lax.axis_index("c")
        base = wid * bw
        pltpu.sync_copy(idx_hbm.at[pl.ds(base, bw)], ids_v)
        rows_lo = lax.iota(jnp.int32, 16)

        def chunk_body(c, carry):
            i0 = c * CHUNK
            def idx_body(q, carry2):
                s = ids_v[pl.ds(i0 + q * 16, 16)]
                b = (s >> 7) * 64 + ((s >> 4) & 7)
                for f in range(NX):
                    off = (f // 8) * FR_G + (f % 8) * 8
                    gi_v[pl.ds(f * CHUNK + q * 16, 16)] = b + off
                return carry2

            lax.fori_loop(0, CHUNK // 16, idx_body, 0)
            pltpu.async_copy(rg_hbm.at[gi_v], rows_v, sem).wait()

            def ext_body(q, carry2):
                s = ids_v[pl.ds(i0 + q * 16, 16)]
                for j in range(16):
                    lane = jnp.broadcast_to(s[j] & 15, (16,))
                    i = q * 16 + j
                    col = jnp.broadcast_to(i0 + i, (16,))
                    lo = plsc.load_gather(
                        rows_v, [rows_lo * CHUNK + i, lane]
                    )
                    hi = plsc.load_gather(
                        rows_v, [(rows_lo + 16) * CHUNK + i, lane]
                    )
                    plsc.store_scatter(vals_v, [rows_lo, col], lo)
                    plsc.store_scatter(vals_v, [rows_lo + 16, col], hi)
                return carry2

            lax.fori_loop(0, CHUNK // 16, ext_body, 0)
            return carry

        lax.fori_loop(0, nch, chunk_body, 0)
        pltpu.sync_copy(vals_v, out_hbm.at[:, pl.ds(base, bw)])

    tt = start_state.T
    rg = _detile(tt).reshape(NG, 16)
    out_t = _gather(rg, ids.astype(jnp.int32))
    return out_t.T


# speedup vs baseline: 3.8560x; 1.1273x over previous
import functools

import jax
import jax.numpy as jnp
from jax import lax
from jax.experimental import pallas as pl
from jax.experimental.pallas import tpu as pltpu
from jax.experimental.pallas import tpu_sc as plsc

NSAMPLES = 1000000
NX = 32
BATCH = 16384

SC_TILES = (NSAMPLES + 127) // 128
KT = 601
R_ROWS = 4 * SC_TILES * 8
NW_F32 = R_ROWS * 128
FR_W = SC_TILES * 1024


def _detile(tt):
    def body(in_ref, out_ref):
        x = in_ref[...]
        out_ref[...] = (
            x.reshape(8, KT, 128).swapaxes(0, 1).reshape(KT * 8, 128)
        )

    return pl.pallas_call(
        body,
        grid=(4, SC_TILES // KT),
        in_specs=[
            pl.BlockSpec((8, KT * 128), lambda fr, g: (fr, g)),
        ],
        out_specs=pl.BlockSpec((KT * 8, 128), lambda fr, g: (fr * (SC_TILES // KT) + g, 0)),
        out_shape=jax.ShapeDtypeStruct((R_ROWS, 128), jnp.float32),
    )(tt)


def kernel(ids, start_state):
    info = plsc.get_sparse_core_info()
    nc, ns = info.num_cores, info.num_subcores
    nw = nc * ns
    bw = BATCH // nw

    mesh = plsc.VectorSubcoreMesh(core_axis_name="c", subcore_axis_name="s")

    @functools.partial(
        pl.kernel,
        mesh=mesh,
        out_type=jax.ShapeDtypeStruct((NX, BATCH), jnp.float32),
        compiler_params=pltpu.CompilerParams(
            use_tc_tiling_on_sc=False, needs_layout_passes=False
        ),
        scratch_types=[
            pltpu.VMEM((bw,), jnp.int32),
            pltpu.VMEM((NX * bw,), jnp.int32),
            pltpu.VMEM((NX * bw,), jnp.float32),
            pltpu.SemaphoreType.DMA,
        ],
    )
    def _gather(rw_hbm, idx_hbm, out_hbm, ids_v, wi_v, vals_v, sem):
        wid = lax.axis_index("s") * nc + lax.axis_index("c")
        base = wid * bw
        pltpu.sync_copy(idx_hbm.at[pl.ds(base, bw)], ids_v)

        def idx_body(q, carry):
            s = ids_v[pl.ds(q * 16, 16)]
            b = (s >> 7) * 1024 + (s & 127)
            for f in range(NX):
                off = (f // 8) * FR_W + (f % 8) * 128
                wi_v[pl.ds(f * bw + q * 16, 16)] = b + off
            return carry

        lax.fori_loop(0, bw // 16, idx_body, 0)
        pltpu.async_copy(rw_hbm.at[wi_v], vals_v, sem).wait()
        for f in range(NX):
            pltpu.sync_copy(
                vals_v.at[pl.ds(f * bw, bw)], out_hbm.at[f, pl.ds(base, bw)]
            )

    tt = start_state.T
    rw = _detile(tt).reshape(NW_F32)
    out_t = _gather(rw, ids.astype(jnp.int32))
    return out_t.T


# speedup vs baseline: 3.9385x; 1.0214x over previous
import functools

import jax
import jax.numpy as jnp
from jax import lax
from jax.experimental import pallas as pl
from jax.experimental.pallas import tpu as pltpu
from jax.experimental.pallas import tpu_sc as plsc

NSAMPLES = 1000000
NX = 32
BATCH = 16384

SC_TILES = (NSAMPLES + 127) // 128
KT = 601
R_ROWS = 4 * SC_TILES * 8
NW_F32 = R_ROWS * 128
FR_W = SC_TILES * 1024


def _detile_octet(tt, fr):
    def body(in_ref, out_ref):
        x = in_ref[...]
        out_ref[...] = (
            x.reshape(8, KT, 128).swapaxes(0, 1).reshape(KT * 8, 128)
        )

    return pl.pallas_call(
        body,
        grid=(SC_TILES // KT,),
        in_specs=[
            pl.BlockSpec((8, KT * 128), lambda g: (fr, g)),
        ],
        out_specs=pl.BlockSpec((KT * 8, 128), lambda g: (g, 0)),
        out_shape=jax.ShapeDtypeStruct((SC_TILES * 8, 128), jnp.float32),
    )(tt)


def kernel(ids, start_state):
    info = plsc.get_sparse_core_info()
    nc, ns = info.num_cores, info.num_subcores
    nw = nc * ns
    bw = BATCH // nw

    mesh = plsc.VectorSubcoreMesh(core_axis_name="c", subcore_axis_name="s")

    @functools.partial(
        pl.kernel,
        mesh=mesh,
        out_type=jax.ShapeDtypeStruct((8, BATCH), jnp.float32),
        compiler_params=pltpu.CompilerParams(
            use_tc_tiling_on_sc=False, needs_layout_passes=False
        ),
        scratch_types=[
            pltpu.VMEM((bw,), jnp.int32),
            pltpu.VMEM((8 * bw,), jnp.int32),
            pltpu.VMEM((8 * bw,), jnp.float32),
            pltpu.SemaphoreType.DMA,
        ],
    )
    def _gather8(rw_hbm, idx_hbm, out_hbm, ids_v, wi_v, vals_v, sem):
        wid = lax.axis_index("s") * nc + lax.axis_index("c")
        base = wid * bw
        pltpu.sync_copy(idx_hbm.at[pl.ds(base, bw)], ids_v)

        def idx_body(q, carry):
            s = ids_v[pl.ds(q * 16, 16)]
            b = (s >> 7) * 1024 + (s & 127)
            for fi in range(8):
                wi_v[pl.ds(fi * bw + q * 16, 16)] = b + fi * 128
            return carry

        lax.fori_loop(0, bw // 16, idx_body, 0)
        pltpu.async_copy(rw_hbm.at[wi_v], vals_v, sem).wait()
        for fi in range(8):
            pltpu.sync_copy(
                vals_v.at[pl.ds(fi * bw, bw)], out_hbm.at[fi, pl.ds(base, bw)]
            )

    tt = start_state.T
    ids32 = ids.astype(jnp.int32)
    outs = []
    for fr in range(4):
        rw = _detile_octet(tt, fr).reshape(FR_W)
        outs.append(_gather8(rw, ids32))
    return jnp.concatenate(outs, axis=0).T
